# baseline (speedup 1.0000x reference)
import functools
import math

import jax
import jax.numpy as jnp
from jax import lax
from jax.experimental import pallas as pl
from jax.experimental.pallas import tpu as pltpu
from jax.experimental.pallas import tpu_sc as plsc

_SMOOTHING = 0.1
_CONF = 1.0 - _SMOOTHING
_PAD = 0

_BR = 256
_BC = 3200
_L = 16


def _tc_body(t_ref, x_ref, o_ref, *, ni):
    i = pl.program_id(0)
    j = pl.program_id(1)
    xb = x_ref[...]
    tgt = t_ref[...]
    rs = jnp.sum(xb, axis=1, keepdims=True)

    @pl.when((i == 0) & (j == 0))
    def _init():
        o_ref[...] = jnp.zeros((ni, 1), jnp.float32)

    first = (j == 0).astype(jnp.float32)
    val = jnp.sum(jnp.where(tgt != _PAD, rs - first * xb[:, 0:1], 0.0))
    rowid = jax.lax.broadcasted_iota(jnp.int32, (ni, 1), 0)
    o_ref[...] += jnp.where(rowid == i, val, 0.0)


def _tc_pass(x, tgt2d):
    n, v = x.shape
    br, bc = _BR, _BC
    ni = n // br
    return pl.pallas_call(
        functools.partial(_tc_body, ni=ni),
        grid=(ni, v // bc),
        in_specs=[
            pl.BlockSpec((br, 1), lambda i, j: (i, 0)),
            pl.BlockSpec((br, bc), lambda i, j: (i, j)),
        ],
        out_specs=pl.BlockSpec((ni, 1), lambda i, j: (0, 0)),
        out_shape=jax.ShapeDtypeStruct((ni, 1), jnp.float32),
        compiler_params=pltpu.CompilerParams(
            dimension_semantics=("arbitrary", "arbitrary"),
        ),
    )(tgt2d, x)


def _make_sc_gather(n, w, nc, nw, sval, kconst):
    b_per_w = n // nw
    ngroups = b_per_w // _L
    mesh = plsc.VectorSubcoreMesh(core_axis_name="c", subcore_axis_name="s")

    @functools.partial(
        pl.kernel,
        mesh=mesh,
        out_type=jax.ShapeDtypeStruct((nw, _L), jnp.float32),
        scratch_types=[
            pltpu.VMEM((b_per_w,), jnp.int32),
            pltpu.VMEM((b_per_w,), jnp.int32),
            pltpu.VMEM((b_per_w, 128), jnp.float32),
            pltpu.VMEM((_L,), jnp.float32),
            pltpu.SemaphoreType.DMA,
        ],
        compiler_params=pltpu.CompilerParams(needs_layout_passes=False),
    )
    def sc_gather(x3_hbm, tgt_hbm, out_hbm, tgt_v, idx_v, rows_v, acc_v, sem):
        wid = lax.axis_index("s") * nc + lax.axis_index("c")
        base = wid * b_per_w
        pltpu.sync_copy(tgt_hbm.at[pl.ds(base, b_per_w)], tgt_v)
        i16 = lax.iota(jnp.int32, _L)
        for g in range(ngroups):
            t16 = tgt_v[pl.ds(g * _L, _L)]
            win = lax.shift_right_logical(t16, 7)
            rowi = (base + g * _L) + i16
            idx_v[pl.ds(g * _L, _L)] = rowi * w + win
        pltpu.async_copy(x3_hbm.at[idx_v], rows_v, sem).wait()
        acc = jnp.zeros((_L,), jnp.float32)
        for g in range(ngroups):
            t16 = tgt_v[pl.ds(g * _L, _L)]
            tmod = jnp.bitwise_and(t16, 127)
            xt = plsc.load_gather(rows_v, [i16 + g * _L, tmod])
            acc = acc + jnp.where(
                t16 != _PAD, kconst + (sval - _CONF) * xt, 0.0)
        acc_v[...] = acc
        pltpu.sync_copy(acc_v, out_hbm.at[wid])

    return sc_gather


def kernel(x, target):
    n, v = x.shape
    w = v // 128
    sval = _SMOOTHING / (v - 2)
    kconst = (v - 2) * sval * math.log(sval) + _CONF * math.log(_CONF)

    info = plsc.get_sparse_core_info()
    nc = info.num_cores
    nw = nc * info.num_subcores

    tgt = target.astype(jnp.int32)
    x3 = x.reshape(n * w, 128)
    parts_sc = _make_sc_gather(n, w, nc, nw, sval, kconst)(x3, tgt)
    parts_tc = _tc_pass(x, tgt[:, None])
    return jnp.sum(parts_sc) - sval * jnp.sum(parts_tc)


# speedup vs baseline: 3.0249x; 3.0249x over previous
import functools
import math

import jax
import jax.numpy as jnp
from jax import lax
from jax.experimental import pallas as pl
from jax.experimental.pallas import tpu as pltpu
from jax.experimental.pallas import tpu_sc as plsc

_SMOOTHING = 0.1
_CONF = 1.0 - _SMOOTHING
_PAD = 0

_BR = 256
_BC = 3200
_L = 16
_NT = 2560
_CH = 640


def _tc_body(t_ref, x_ref, o_ref, *, ni, bc, sval, kconst):
    i = pl.program_id(0)
    j = pl.program_id(1)
    xb = x_ref[...]
    tgt = t_ref[...]
    br = xb.shape[0]
    m = (tgt != _PAD).astype(jnp.float32)
    rs = jnp.sum(xb, axis=1, keepdims=True)
    colid = jax.lax.broadcasted_iota(jnp.int32, (br, bc), 1)
    hit = colid == (tgt - j * bc)
    xts = jnp.sum(jnp.where(hit, xb, 0.0), axis=1, keepdims=True)
    sj = jnp.sum(m * ((sval - _CONF) * xts - sval * rs))
    sj += jnp.where(
        j == 0, jnp.sum(m * (sval * xb[:, 0:1] + kconst)), 0.0)

    @pl.when((i == 0) & (j == 0))
    def _init():
        o_ref[...] = jnp.zeros((ni, 1), jnp.float32)

    rowid = jax.lax.broadcasted_iota(jnp.int32, (ni, 1), 0)
    o_ref[...] += jnp.where(rowid == i, sj, 0.0)


def _tc_pass(x, tgt2d, nt, sval, kconst):
    _, v = x.shape
    br, bc = _BR, _BC
    ni = nt // br
    return pl.pallas_call(
        functools.partial(_tc_body, ni=ni, bc=bc, sval=sval, kconst=kconst),
        grid=(ni, v // bc),
        in_specs=[
            pl.BlockSpec((br, 1), lambda i, j: (i, 0)),
            pl.BlockSpec((br, bc), lambda i, j: (i, j)),
        ],
        out_specs=pl.BlockSpec((ni, 1), lambda i, j: (0, 0)),
        out_shape=jax.ShapeDtypeStruct((ni, 1), jnp.float32),
        compiler_params=pltpu.CompilerParams(
            dimension_semantics=("arbitrary", "arbitrary"),
        ),
    )(tgt2d, x)


def _make_sc_stream(n, v, nt, nc, nw, sval, kconst):
    rows_sc = n - nt
    r_w = rows_sc // nw
    ngroups = r_w // _L
    nch = v // _CH
    assert nch % 2 == 0
    mesh = plsc.VectorSubcoreMesh(core_axis_name="c", subcore_axis_name="s")

    @functools.partial(
        pl.kernel,
        mesh=mesh,
        out_type=jax.ShapeDtypeStruct((nw, _L), jnp.float32),
        scratch_types=[
            pltpu.VMEM((_L,), jnp.int32),
            pltpu.VMEM((_L, _CH), jnp.float32),
            pltpu.VMEM((_L, _CH), jnp.float32),
            pltpu.VMEM((_L, _L), jnp.float32),
            pltpu.VMEM((_L,), jnp.float32),
            pltpu.SemaphoreType.DMA,
            pltpu.SemaphoreType.DMA,
        ],
        compiler_params=pltpu.CompilerParams(needs_layout_passes=False),
    )
    def sc_stream(x_hbm, tgt_hbm, out_hbm, tgt_v, buf0, buf1, red_v,
                  acc_v, sem0, sem1):
        wid = lax.axis_index("s") * nc + lax.axis_index("c")
        base_row = nt + wid * r_w
        i16 = lax.iota(jnp.int32, _L)
        z16 = jnp.zeros((_L,), jnp.int32)
        lane15 = z16 + 15
        total = jnp.zeros((_L,), jnp.float32)

        for g in range(ngroups):
            r0 = base_row + g * _L
            pltpu.sync_copy(tgt_hbm.at[pl.ds(r0, _L)], tgt_v)
            t16 = tgt_v[...]
            pltpu.async_copy(
                x_hbm.at[pl.ds(r0, _L), pl.ds(0, _CH)], buf0, sem0)
            pltpu.async_copy(
                x_hbm.at[pl.ds(r0, _L), pl.ds(_CH, _CH)], buf1, sem1)

            carry0 = tuple(jnp.zeros((_L,), jnp.float32)
                           for _ in range(_L + 2))

            def pair_body(ip, carry, *, _r0=r0, _t16=t16):
                for b, (buf, sem) in enumerate(
                        ((buf0, sem0), (buf1, sem1))):
                    ci = ip * 2 + b
                    pltpu.make_async_copy(
                        x_hbm.at[pl.ds(_r0, _L),
                                 pl.ds(ci * _CH, _CH)],
                        buf, sem).wait()

                    accs = carry[:_L]
                    xt_acc = carry[_L]
                    x0_acc = carry[_L + 1]

                    def chunk_body(k, a, *, _buf=buf):
                        off = k * _L
                        return tuple(
                            a[l] + _buf[l, pl.ds(off, _L)]
                            for l in range(_L)
                        )

                    accs = lax.fori_loop(0, _CH // _L, chunk_body, accs)

                    civ = z16 + ci
                    rel = _t16 - civ * _CH
                    inwin = (rel >= 0) & (rel < _CH)
                    idx = jnp.clip(rel, 0, _CH - 1)
                    gath = plsc.load_gather(buf, [i16, idx])
                    xt_acc = xt_acc + jnp.where(inwin, gath, 0.0)
                    g0 = plsc.load_gather(buf, [i16, z16])
                    x0_acc = x0_acc + jnp.where(civ == 0, g0, 0.0)

                    @pl.when(ci + 2 < nch)
                    def _refill(*, _buf=buf, _sem=sem, _ci=ci):
                        pltpu.async_copy(
                            x_hbm.at[pl.ds(_r0, _L),
                                     pl.ds((_ci + 2) * _CH, _CH)],
                            _buf, _sem)

                    carry = accs + (xt_acc, x0_acc)
                return carry

            carry = lax.fori_loop(0, nch // 2, pair_body, carry0)
            for l in range(_L):
                red_v[l] = plsc.cumsum(carry[l])
            rs_all = plsc.load_gather(red_v, [i16, lane15])
            xt = carry[_L]
            x0 = carry[_L + 1]
            kl = kconst - sval * (rs_all - x0) + (sval - _CONF) * xt
            total = total + jnp.where(t16 != _PAD, kl, 0.0)

        acc_v[...] = total
        pltpu.sync_copy(acc_v, out_hbm.at[wid])

    return sc_stream


def kernel(x, target):
    n, v = x.shape
    sval = _SMOOTHING / (v - 2)
    kconst = (v - 2) * sval * math.log(sval) + _CONF * math.log(_CONF)

    info = plsc.get_sparse_core_info()
    nc = info.num_cores
    nw = nc * info.num_subcores

    tgt = target.astype(jnp.int32)
    nt = _NT
    parts_sc = _make_sc_stream(n, v, nt, nc, nw, sval, kconst)(x, tgt)
    parts_tc = _tc_pass(x, tgt[:nt, None], nt, sval, kconst)
    return jnp.sum(parts_tc) + jnp.sum(parts_sc)
